# baseline (speedup 1.0000x reference)
import functools

import jax
import jax.numpy as jnp
from jax import lax
from jax.experimental import pallas as pl
from jax.experimental.pallas import tpu as pltpu
from jax.experimental.pallas import tpu_sc as plsc

NCORE = 2
NSUB = 16
NW = NCORE * NSUB
LANES = 16
D = 64
W = 2 * D
CHUNK = 128
KNEG = 8
MARGIN = 1.0
NPOS = 4096


def _tec_body(h_hbm, t_hbm, r_hbm, ent_hbm, rel_hbm, out_hbm,
              idx_h, idx_t, idx_r, pidx_h, pidx_t, pidx_r,
              rows_h, rows_t, rows_r,
              scores_p, scores_n, loss_buf, sem):
    cid = lax.axis_index("c")
    sid = lax.axis_index("s")
    wid = sid * NCORE + cid
    lane = lax.iota(jnp.int32, LANES)

    def stage_chunk(base):
        pltpu.sync_copy(h_hbm.at[pl.ds(base, CHUNK)], idx_h)
        pltpu.sync_copy(t_hbm.at[pl.ds(base, CHUNK)], idx_t)
        pltpu.sync_copy(r_hbm.at[pl.ds(base, CHUNK)], idx_r)

        def pair_body(g, carry):
            row0 = g * LANES + lane
            plsc.store_scatter(pidx_h, [row0],
                               plsc.load_gather(idx_h, [row0]) >> 1)
            plsc.store_scatter(pidx_t, [row0],
                               plsc.load_gather(idx_t, [row0]) >> 1)
            plsc.store_scatter(pidx_r, [row0],
                               plsc.load_gather(idx_r, [row0]) >> 1)
            return carry

        lax.fori_loop(0, CHUNK // LANES, pair_body, jnp.int32(0))
        pltpu.async_copy(ent_hbm.at[pidx_h], rows_h, sem).wait()
        pltpu.async_copy(ent_hbm.at[pidx_t], rows_t, sem).wait()
        pltpu.async_copy(rel_hbm.at[pidx_r], rows_r, sem).wait()

    def score_chunk(scores_ref):
        def g_body(g, carry):
            row0 = g * LANES + lane
            cb_h = (plsc.load_gather(idx_h, [row0]) & 1) * D
            cb_t = (plsc.load_gather(idx_t, [row0]) & 1) * D
            cb_r = (plsc.load_gather(idx_r, [row0]) & 1) * D

            def d_body(dd, acc):
                hv = plsc.load_gather(rows_h, [row0, cb_h + dd])
                rv = plsc.load_gather(rows_r, [row0, cb_r + dd])
                tv = plsc.load_gather(rows_t, [row0, cb_t + dd])
                return acc + jnp.abs(hv + rv - tv)

            acc = lax.fori_loop(0, D, d_body, jnp.zeros((LANES,), jnp.float32),
                                unroll=8)
            plsc.store_scatter(scores_ref, [row0], acc)
            return carry

        lax.fori_loop(0, CHUNK // LANES, g_body, jnp.int32(0))

    stage_chunk(wid * CHUNK)
    score_chunk(scores_p)

    loss_acc = jnp.zeros((LANES,), jnp.float32)
    for j in range(KNEG):
        stage_chunk(NPOS + wid * (CHUNK * KNEG) + j * CHUNK)
        score_chunk(scores_n)
        nacc = jnp.zeros((LANES,), jnp.float32)
        for k in range(KNEG):
            nacc = nacc + plsc.load_gather(scores_n, [lane * KNEG + k])
        p = scores_p[pl.ds(j * LANES, LANES)]
        loss_acc = loss_acc + jnp.maximum(p - nacc * (1.0 / KNEG) + MARGIN, 0.0)

    loss_buf[...] = loss_acc
    pltpu.sync_copy(loss_buf, out_hbm.at[pl.ds(wid * LANES, LANES)])


@jax.jit
def kernel(batch_h, batch_t, batch_r, batch_size, n_negative,
           ent_embeddings, rel_embeddings):
    del batch_size, n_negative
    n_ent = ent_embeddings.shape[0]
    n_rel = rel_embeddings.shape[0]
    ent2 = ent_embeddings.reshape(n_ent // 2, W)
    rel2 = rel_embeddings.reshape(n_rel // 2, W)
    mesh = plsc.VectorSubcoreMesh(core_axis_name="c", subcore_axis_name="s",
                                  num_cores=NCORE, num_subcores=NSUB)
    kern = pl.kernel(
        _tec_body,
        out_type=jax.ShapeDtypeStruct((NW * LANES,), jnp.float32),
        mesh=mesh,
        compiler_params=pltpu.CompilerParams(needs_layout_passes=False,
                                             use_tc_tiling_on_sc=True),
        scratch_types=[
            pltpu.VMEM((CHUNK,), jnp.int32),
            pltpu.VMEM((CHUNK,), jnp.int32),
            pltpu.VMEM((CHUNK,), jnp.int32),
            pltpu.VMEM((CHUNK,), jnp.int32),
            pltpu.VMEM((CHUNK,), jnp.int32),
            pltpu.VMEM((CHUNK,), jnp.int32),
            pltpu.VMEM((CHUNK, W), jnp.float32),
            pltpu.VMEM((CHUNK, W), jnp.float32),
            pltpu.VMEM((CHUNK, W), jnp.float32),
            pltpu.VMEM((CHUNK,), jnp.float32),
            pltpu.VMEM((CHUNK,), jnp.float32),
            pltpu.VMEM((LANES,), jnp.float32),
            pltpu.SemaphoreType.DMA,
        ],
    )
    partials = kern(batch_h, batch_t, batch_r, ent2, rel2)
    return jnp.sum(partials)


# speedup vs baseline: 1.2803x; 1.2803x over previous
import jax
import jax.numpy as jnp
from jax import lax
from jax.experimental import pallas as pl
from jax.experimental.pallas import tpu as pltpu
from jax.experimental.pallas import tpu_sc as plsc

NCORE = 2
NSUB = 16
NW = NCORE * NSUB
LANES = 16
D = 64
W = 2 * D
KNEG = 8
MARGIN = 1.0
NPOS = 4096
TOTAL = 36864
N_ENT = 1000000

EB = 1024
NSB_FULL = N_ENT // EB
TAIL = N_ENT - NSB_FULL * EB
RAGGED = N_ENT % 128
ALIGNED_TAIL = TAIL - RAGGED
SB_ITERS = (NSB_FULL + NW) // NW
CAP = 6144
IDXC = 4096
NSCAN = TOTAL // IDXC
G_ROWS = 2 * TOTAL + 128
CHUNK = 128


def _extract_body(h_hbm, t_hbm, entT_hbm, tail_hbm, g_hbm,
                  idxbuf, me, ms, sb_e, sb_s, blk, tailbuf, extbuf, sidx, sem):
    cid = lax.axis_index("c")
    sid = lax.axis_index("s")
    wid = sid * NCORE + cid
    lane = lax.iota(jnp.int32, LANES)

    def scan_chunk(src_hbm, slot0, pass_base, cnt, total):
        def v_body(v, carry):
            cnt, total = carry
            ev = plsc.load_gather(idxbuf, [v * LANES + lane])
            m = ((ev >> 10) & (NW - 1)) == wid
            mi = m.astype(jnp.int32)
            rank = plsc.cumsum(mi)
            g_idx = total + rank - 1
            keep = m & (g_idx >= pass_base) & (g_idx < pass_base + CAP)
            nk = plsc.all_reduce_population_count(keep)[0]
            nm = plsc.all_reduce_population_count(m)[0]
            plsc.store_compressed(me.at[pl.ds(cnt, LANES)], ev, mask=keep)
            plsc.store_compressed(ms.at[pl.ds(cnt, LANES)],
                                  slot0 + v * LANES + lane, mask=keep)
            return cnt + nk, total + nm

        return lax.fori_loop(0, IDXC // LANES, v_body, (cnt, total))

    def do_scan(pass_base):
        cnt = jnp.int32(0)
        total = jnp.int32(0)
        for part, src in ((0, h_hbm), (1, t_hbm)):
            def c_body(ic, carry, part=part, src=src):
                cnt, total = carry
                pltpu.sync_copy(src.at[pl.ds(ic * IDXC, IDXC)], idxbuf)
                return scan_chunk(src, part * TOTAL + ic * IDXC, pass_base,
                                  cnt, total)
            cnt, total = lax.fori_loop(0, NSCAN, c_body, (cnt, total))
        return cnt, total

    def extract_subblock(sb, e_base, cnt):
        def f_body(v, fcnt):
            idx = v * LANES + lane
            valid = idx < cnt
            ev = plsc.load_gather(me, [jnp.minimum(idx, CAP - 1)], mask=valid)
            sv = plsc.load_gather(ms, [jnp.minimum(idx, CAP - 1)], mask=valid)
            m = valid & ((ev >> 10) == sb)
            nk = plsc.all_reduce_population_count(m)[0]
            plsc.store_compressed(sb_e.at[pl.ds(fcnt, LANES)], ev, mask=m)
            plsc.store_compressed(sb_s.at[pl.ds(fcnt, LANES)], sv, mask=m)
            return fcnt + nk

        nvr = (cnt + LANES - 1) // LANES
        fcnt = lax.fori_loop(0, nvr, f_body, jnp.int32(0))

        def b_body(b, carry):
            for g16 in range(CHUNK // LANES):
                idx = b * CHUNK + g16 * LANES + lane
                valid = idx < fcnt
                cidx = jnp.minimum(idx, CAP - 1)
                e16 = plsc.load_gather(sb_e, [cidx], mask=valid)
                s16 = plsc.load_gather(sb_s, [cidx], mask=valid)
                col = jnp.where(valid, e16 - e_base, 0)
                slot = jnp.where(valid, s16, 2 * TOTAL + g16 * LANES + lane)
                plsc.store_scatter(sidx, [g16 * LANES + lane], slot)
                row = jnp.full((LANES,), g16 * LANES, jnp.int32) + lane

                def d_body(dd, c):
                    dfull = jnp.full((LANES,), dd, jnp.int32)
                    v = plsc.load_gather(blk, [dfull, col])
                    plsc.store_scatter(extbuf, [row, dfull], v)
                    return c

                lax.fori_loop(0, D, d_body, jnp.int32(0), unroll=8)
            pltpu.async_copy(extbuf, g_hbm.at[sidx], sem).wait()
            return carry

        nb = (fcnt + CHUNK - 1) // CHUNK
        lax.fori_loop(0, nb, b_body, jnp.int32(0))

    def do_extract(cnt):
        def sb_body(i, carry):
            sb = wid + i * NW

            @pl.when(sb < NSB_FULL)
            def _():
                pltpu.sync_copy(entT_hbm.at[:, pl.ds(sb * EB, EB)], blk)
                extract_subblock(sb, sb * EB, cnt)

            @pl.when(sb == NSB_FULL)
            def _():
                pltpu.sync_copy(entT_hbm.at[:, pl.ds(NSB_FULL * EB, ALIGNED_TAIL)],
                                blk.at[:, pl.ds(0, ALIGNED_TAIL)])
                pltpu.sync_copy(tail_hbm, tailbuf)

                def tr_body(dd, carry):
                    dfull = jnp.full((LANES,), dd, jnp.int32)
                    for jg in range(RAGGED // LANES):
                        j16 = jnp.full((LANES,), jg * LANES, jnp.int32) + lane
                        v = plsc.load_gather(tailbuf, [j16, dfull])
                        plsc.store_scatter(blk, [dfull, ALIGNED_TAIL + j16], v)
                    return carry

                lax.fori_loop(0, D, tr_body, jnp.int32(0))
                extract_subblock(sb, NSB_FULL * EB, cnt)

            return carry

        lax.fori_loop(0, SB_ITERS, sb_body, jnp.int32(0))

    def p_cond(state):
        pass_base, total = state
        return (pass_base == 0) | (pass_base < total)

    def p_body(state):
        pass_base, _ = state
        cnt, total = do_scan(pass_base)
        do_extract(cnt)
        return pass_base + CAP, total

    lax.while_loop(p_cond, p_body, (jnp.int32(0), jnp.int32(0)))


def _score_body(r_hbm, g_hbm, rel_hbm, out_hbm,
                idx_r, pidx_r, rows_h, rows_t, rows_r,
                scores_p, scores_n, loss_buf, sem):
    cid = lax.axis_index("c")
    sid = lax.axis_index("s")
    wid = sid * NCORE + cid
    lane = lax.iota(jnp.int32, LANES)

    def stage_chunk(base):
        pltpu.sync_copy(g_hbm.at[pl.ds(base, CHUNK)], rows_h)
        pltpu.sync_copy(g_hbm.at[pl.ds(TOTAL + base, CHUNK)], rows_t)
        pltpu.sync_copy(r_hbm.at[pl.ds(base, CHUNK)], idx_r)

        def pair_body(g, carry):
            row0 = g * LANES + lane
            plsc.store_scatter(pidx_r, [row0],
                               plsc.load_gather(idx_r, [row0]) >> 1)
            return carry

        lax.fori_loop(0, CHUNK // LANES, pair_body, jnp.int32(0))
        pltpu.async_copy(rel_hbm.at[pidx_r], rows_r, sem).wait()

    def score_chunk(scores_ref):
        def g_body(g, carry):
            row0 = g * LANES + lane
            cb_r = (plsc.load_gather(idx_r, [row0]) & 1) * D

            def d_body(dd, acc):
                hv = plsc.load_gather(rows_h, [row0, jnp.full((LANES,), dd,
                                                              jnp.int32)])
                rv = plsc.load_gather(rows_r, [row0, cb_r + dd])
                tv = plsc.load_gather(rows_t, [row0, jnp.full((LANES,), dd,
                                                              jnp.int32)])
                return acc + jnp.abs(hv + rv - tv)

            acc = lax.fori_loop(0, D, d_body, jnp.zeros((LANES,), jnp.float32),
                                unroll=8)
            plsc.store_scatter(scores_ref, [row0], acc)
            return carry

        lax.fori_loop(0, CHUNK // LANES, g_body, jnp.int32(0))

    stage_chunk(wid * CHUNK)
    score_chunk(scores_p)

    loss_acc = jnp.zeros((LANES,), jnp.float32)
    for j in range(KNEG):
        stage_chunk(NPOS + wid * (CHUNK * KNEG) + j * CHUNK)
        score_chunk(scores_n)
        nacc = jnp.zeros((LANES,), jnp.float32)
        for k in range(KNEG):
            nacc = nacc + plsc.load_gather(scores_n, [lane * KNEG + k])
        p = scores_p[pl.ds(j * LANES, LANES)]
        loss_acc = loss_acc + jnp.maximum(p - nacc * (1.0 / KNEG) + MARGIN, 0.0)

    loss_buf[...] = loss_acc
    pltpu.sync_copy(loss_buf, out_hbm.at[pl.ds(wid * LANES, LANES)])


@jax.jit
def kernel(batch_h, batch_t, batch_r, batch_size, n_negative,
           ent_embeddings, rel_embeddings):
    del batch_size, n_negative
    entT = ent_embeddings.T
    rel2 = rel_embeddings.reshape(rel_embeddings.shape[0] // 2, W)
    mesh = plsc.VectorSubcoreMesh(core_axis_name="c", subcore_axis_name="s",
                                  num_cores=NCORE, num_subcores=NSUB)
    params = pltpu.CompilerParams(needs_layout_passes=False,
                                  use_tc_tiling_on_sc=True)

    kern1 = pl.kernel(
        _extract_body,
        out_type=jax.ShapeDtypeStruct((G_ROWS, W), jnp.float32),
        mesh=mesh,
        compiler_params=params,
        scratch_types=[
            pltpu.VMEM((IDXC,), jnp.int32),
            pltpu.VMEM((CAP + LANES,), jnp.int32),
            pltpu.VMEM((CAP + LANES,), jnp.int32),
            pltpu.VMEM((CAP + LANES,), jnp.int32),
            pltpu.VMEM((CAP + LANES,), jnp.int32),
            pltpu.VMEM((D, EB), jnp.float32),
            pltpu.VMEM((RAGGED, D), jnp.float32),
            pltpu.VMEM((CHUNK, W), jnp.float32),
            pltpu.VMEM((CHUNK,), jnp.int32),
            pltpu.SemaphoreType.DMA,
        ],
    )
    tail_rows = lax.slice(ent_embeddings, (N_ENT - RAGGED, 0), (N_ENT, D))
    g = kern1(batch_h, batch_t, entT, tail_rows)

    kern2 = pl.kernel(
        _score_body,
        out_type=jax.ShapeDtypeStruct((NW * LANES,), jnp.float32),
        mesh=mesh,
        compiler_params=params,
        scratch_types=[
            pltpu.VMEM((CHUNK,), jnp.int32),
            pltpu.VMEM((CHUNK,), jnp.int32),
            pltpu.VMEM((CHUNK, W), jnp.float32),
            pltpu.VMEM((CHUNK, W), jnp.float32),
            pltpu.VMEM((CHUNK, W), jnp.float32),
            pltpu.VMEM((CHUNK,), jnp.float32),
            pltpu.VMEM((CHUNK,), jnp.float32),
            pltpu.VMEM((LANES,), jnp.float32),
            pltpu.SemaphoreType.DMA,
        ],
    )
    partials = kern2(batch_r, g, rel2)
    return jnp.sum(partials)
